# baseline (speedup 1.0000x reference)
import functools

import jax
import jax.numpy as jnp
from jax import lax
from jax.experimental import pallas as pl
from jax.experimental.pallas import tpu as pltpu
from jax.experimental.pallas import tpu_sc as plsc
from jax.experimental.layout import Format, Layout, with_layout_constraint

_B = 16384
_D_NUM = 13
_N_CAT = 26
_VOCAB = 100000
_D = 64
_NNUM = 1 + _D_NUM
_NROW = _NNUM + _N_CAT

_NC, _NS = 2, 16
_NW = _NC * _NS
_PER_W = _B // _NW
_SB = 16
_NCHUNK = _PER_W // _SB
_GROWS = 104
_GPC = (_SB * _N_CAT) // _GROWS


def _body(idx_hbm, tab_hbm, xnum_hbm, w_hbm, bn_hbm, bc_hbm, out_hbm,
          idx_res, xnum_res, w_res, bn_res, bc_res, cat_stage, stage, sem):
    wid = lax.axis_index("s") * _NC + lax.axis_index("c")

    nidx = _PER_W * _N_CAT
    pltpu.sync_copy(idx_hbm.at[pl.ds(wid * nidx, nidx)], idx_res)
    pltpu.sync_copy(xnum_hbm.at[pl.ds(wid * (_PER_W // 8), _PER_W // 8), :],
                    xnum_res)
    pltpu.sync_copy(w_hbm, w_res)
    pltpu.sync_copy(bn_hbm, bn_res)
    pltpu.sync_copy(bc_hbm, bc_res)

    def chunk_body(c, carry):
        descs = [
            pltpu.async_copy(
                tab_hbm.at[idx_res.at[pl.ds(c * _SB * _N_CAT + gi * _GROWS,
                                            _GROWS)]],
                cat_stage.at[pl.ds(gi * _GROWS, _GROWS), :],
                sem)
            for gi in range(_GPC)
        ]
        for d in descs:
            d.wait()

        def row_body(b_off, carry2):
            crow = b_off * _N_CAT
            srow = b_off * (_NROW // 2)
            bloc = c * _SB + b_off
            xv = xnum_res[bloc >> 3, pl.ds((bloc & 7) * 16, 16)]
            for j in range(_NNUM):
                x = xv[j]
                half = (j % 2) * 64
                for ch in range(4):
                    stage[srow + j // 2, pl.ds(half + ch * 16, 16)] = (
                        w_res[j, pl.ds(ch * 16, 16)] * x
                        + bn_res[j, pl.ds(ch * 16, 16)])
            for i in range(_N_CAT):
                half = ((_NNUM + i) % 2) * 64
                q = (_NNUM + i) // 2
                for ch in range(4):
                    stage[srow + q, pl.ds(half + ch * 16, 16)] = (
                        cat_stage[crow + i, pl.ds(ch * 16, 16)]
                        + bc_res[i, pl.ds(ch * 16, 16)])
            return carry2

        lax.fori_loop(0, _SB, row_body, 0)

        base = (wid * _PER_W + c * _SB) * (_NROW // 2)
        pltpu.sync_copy(stage, out_hbm.at[pl.ds(base, _SB * (_NROW // 2)), :])
        return carry

    lax.fori_loop(0, _NCHUNK, chunk_body, 0)


@jax.jit
def _tokenize(idx, tab, xnf, weight, bn, bc):
    mesh = plsc.VectorSubcoreMesh(
        core_axis_name="c", subcore_axis_name="s",
        num_cores=_NC, num_subcores=_NS)
    f = pl.kernel(
        _body,
        out_type=jax.ShapeDtypeStruct((_B * _NROW // 2, 128), jnp.float32),
        mesh=mesh,
        scratch_types=[
            pltpu.VMEM((_PER_W * _N_CAT,), jnp.int32),
            pltpu.VMEM((_PER_W // 8, 128), jnp.float32),
            pltpu.VMEM((_NNUM, _D), jnp.float32),
            pltpu.VMEM((_NNUM, _D), jnp.float32),
            pltpu.VMEM((_N_CAT, _D), jnp.float32),
            pltpu.VMEM((_SB * _N_CAT, _D), jnp.float32),
            pltpu.VMEM((_SB * _NROW // 2, 128), jnp.float32),
            pltpu.SemaphoreType.DMA,
        ],
        compiler_params=pltpu.CompilerParams(use_tc_tiling_on_sc=False),
    )
    return f(idx, tab, xnf, weight, bn, bc)


_STD3 = Layout(major_to_minor=(0, 1, 2))


def kernel(x_num, x_cat, emb_tables, weight, bias):
    x_cat = x_cat.astype(jnp.int32)
    emb_tables = with_layout_constraint(emb_tables, _STD3)
    idx = (x_cat.T
           + (jnp.arange(_N_CAT, dtype=jnp.int32) * _VOCAB)[None, :]).reshape(-1)
    tab = emb_tables.reshape(_N_CAT * _VOCAB, _D)
    xnf = jnp.concatenate(
        [jnp.ones((_B, 1), x_num.dtype), x_num,
         jnp.zeros((_B, 2), x_num.dtype)], axis=1).reshape(_B // 8, 128)
    bn = jnp.concatenate(
        [jnp.zeros((1, _D), bias.dtype), bias[:_D_NUM]], axis=0)
    bc = bias[_D_NUM:]
    out = _tokenize(idx, tab, xnf, weight, bn, bc)
    return out.reshape(_B, _NROW, _D)


# speedup vs baseline: 1.0438x; 1.0438x over previous
import functools

import jax
import jax.numpy as jnp
from jax import lax
from jax.experimental import pallas as pl
from jax.experimental.pallas import tpu as pltpu
from jax.experimental.pallas import tpu_sc as plsc
from jax.experimental.layout import Format, Layout, with_layout_constraint

_B = 16384
_D_NUM = 13
_N_CAT = 26
_VOCAB = 100000
_D = 64
_NNUM = 1 + _D_NUM
_NROW = _NNUM + _N_CAT

_NC, _NS = 2, 16
_NW = _NC * _NS
_PER_W = _B // _NW
_SB = 8
_NCHUNK = _PER_W // _SB
_GROWS = 104
_GPC = (_SB * _N_CAT) // _GROWS


def _body(idx_hbm, tab_hbm, xnum_hbm, w_hbm, bn_hbm, bc_hbm, out_hbm,
          idx_res, xnum_res, w_res, bn_res, bc_res,
          cat0, cat1, stage0, stage1, sem_g, sem_o):
    wid = lax.axis_index("s") * _NC + lax.axis_index("c")
    cats = (cat0, cat1)
    stages = (stage0, stage1)
    srows = _SB * (_NROW // 2)

    nidx = _PER_W * _N_CAT
    pltpu.sync_copy(idx_hbm.at[pl.ds(wid * nidx, nidx)], idx_res)
    pltpu.sync_copy(xnum_hbm.at[pl.ds(wid * (_PER_W // 8), _PER_W // 8), :],
                    xnum_res)
    pltpu.sync_copy(w_hbm, w_res)
    pltpu.sync_copy(bn_hbm, bn_res)
    pltpu.sync_copy(bc_hbm, bc_res)

    def fire_gather(c, buf):
        cc = lax.rem(c, _NCHUNK)
        for gi in range(_GPC):
            pltpu.async_copy(
                tab_hbm.at[idx_res.at[pl.ds(cc * _SB * _N_CAT + gi * _GROWS,
                                            _GROWS)]],
                buf.at[pl.ds(gi * _GROWS, _GROWS), :],
                sem_g)

    def drain_gather(buf):
        for gi in range(_GPC):
            pltpu.make_async_copy(
                tab_hbm.at[idx_res.at[pl.ds(gi * _GROWS, _GROWS)]],
                buf.at[pl.ds(gi * _GROWS, _GROWS), :],
                sem_g).wait()

    def compute(c, cat_stage, stage):
        def row_body(b_off, carry2):
            crow = b_off * _N_CAT
            srow = b_off * (_NROW // 2)
            bloc = c * _SB + b_off
            xv = xnum_res[bloc >> 3, pl.ds((bloc & 7) * 16, 16)]
            for j in range(_NNUM):
                x = xv[j]
                half = (j % 2) * 64
                for ch in range(4):
                    stage[srow + j // 2, pl.ds(half + ch * 16, 16)] = (
                        w_res[j, pl.ds(ch * 16, 16)] * x
                        + bn_res[j, pl.ds(ch * 16, 16)])
            for i in range(_N_CAT):
                half = ((_NNUM + i) % 2) * 64
                q = (_NNUM + i) // 2
                for ch in range(4):
                    stage[srow + q, pl.ds(half + ch * 16, 16)] = (
                        cat_stage[crow + i, pl.ds(ch * 16, 16)]
                        + bc_res[i, pl.ds(ch * 16, 16)])
            return carry2

        lax.fori_loop(0, _SB, row_body, 0)

    def fire_out(c, stage):
        base = (wid * _PER_W + c * _SB) * (_NROW // 2)
        pltpu.async_copy(stage, out_hbm.at[pl.ds(base, srows), :], sem_o)

    def drain_out(stage):
        pltpu.make_async_copy(
            stage, out_hbm.at[pl.ds(0, srows), :], sem_o).wait()

    fire_gather(0, cats[0])
    drain_gather(cats[0])
    fire_gather(1, cats[1])
    compute(0, cats[0], stages[0])
    fire_out(0, stages[0])

    def chunk_body(c, carry):
        buf = lax.rem(c, 2)

        def even(_):
            drain_gather(cats[1])
            fire_gather(c + 1, cats[0])
            compute(c, cats[1], stages[1])
            drain_out(stages[0])
            fire_out(c, stages[1])
            return 0

        def odd(_):
            drain_gather(cats[0])
            fire_gather(c + 1, cats[1])
            compute(c, cats[0], stages[0])
            drain_out(stages[1])
            fire_out(c, stages[0])
            return 0

        lax.cond(buf == 1, even, odd, 0)
        return carry

    lax.fori_loop(1, _NCHUNK, chunk_body, 0)
    drain_gather(cats[_NCHUNK % 2])
    drain_out(stages[(_NCHUNK - 1) % 2])


@jax.jit
def _tokenize(idx, tab, xnf, weight, bn, bc):
    mesh = plsc.VectorSubcoreMesh(
        core_axis_name="c", subcore_axis_name="s",
        num_cores=_NC, num_subcores=_NS)
    f = pl.kernel(
        _body,
        out_type=jax.ShapeDtypeStruct((_B * _NROW // 2, 128), jnp.float32),
        mesh=mesh,
        scratch_types=[
            pltpu.VMEM((_PER_W * _N_CAT,), jnp.int32),
            pltpu.VMEM((_PER_W // 8, 128), jnp.float32),
            pltpu.VMEM((_NNUM, _D), jnp.float32),
            pltpu.VMEM((_NNUM, _D), jnp.float32),
            pltpu.VMEM((_N_CAT, _D), jnp.float32),
            pltpu.VMEM((_SB * _N_CAT, _D), jnp.float32),
            pltpu.VMEM((_SB * _N_CAT, _D), jnp.float32),
            pltpu.VMEM((_SB * _NROW // 2, 128), jnp.float32),
            pltpu.VMEM((_SB * _NROW // 2, 128), jnp.float32),
            pltpu.SemaphoreType.DMA,
            pltpu.SemaphoreType.DMA,
        ],
        compiler_params=pltpu.CompilerParams(use_tc_tiling_on_sc=False),
    )
    return f(idx, tab, xnf, weight, bn, bc)


_STD3 = Layout(major_to_minor=(0, 1, 2))


def kernel(x_num, x_cat, emb_tables, weight, bias):
    x_cat = x_cat.astype(jnp.int32)
    emb_tables = with_layout_constraint(emb_tables, _STD3)
    idx = (x_cat.T
           + (jnp.arange(_N_CAT, dtype=jnp.int32) * _VOCAB)[None, :]).reshape(-1)
    tab = emb_tables.reshape(_N_CAT * _VOCAB, _D)
    xnf = jnp.concatenate(
        [jnp.ones((_B, 1), x_num.dtype), x_num,
         jnp.zeros((_B, 2), x_num.dtype)], axis=1).reshape(_B // 8, 128)
    bn = jnp.concatenate(
        [jnp.zeros((1, _D), bias.dtype), bias[:_D_NUM]], axis=0)
    bc = bias[_D_NUM:]
    out = _tokenize(idx, tab, xnf, weight, bn, bc)
    return out.reshape(_B, _NROW, _D)
